# baseline (speedup 1.0000x reference)
import jax
import jax.numpy as jnp
from jax.experimental import pallas as pl


_BH = 512


def _body(x_ref, p_ref, o_ref):
    o_ref[...] = x_ref[...] + p_ref[...]


def kernel(input_tensor, position_embeddings):
    B, S, H = input_tensor.shape
    grid = (H // _BH, B)
    return pl.pallas_call(
        _body,
        grid=grid,
        in_specs=[
            pl.BlockSpec((1, S, _BH), lambda h, b: (b, 0, h)),
            pl.BlockSpec((S, _BH), lambda h, b: (0, h)),
        ],
        out_specs=pl.BlockSpec((1, S, _BH), lambda h, b: (b, 0, h)),
        out_shape=jax.ShapeDtypeStruct((B, S, H), input_tensor.dtype),
    )(input_tensor, position_embeddings)
